# baseline (speedup 1.0000x reference)
import functools

import jax
import jax.numpy as jnp
from jax import lax
from jax.experimental import pallas as pl
from jax.experimental.pallas import tpu as pltpu
from jax.experimental.pallas import tpu_sc as plsc

_NCHUNK = 2
_CHUNK = 1600000 // _NCHUNK

_U_WORKERS = 25
_U_LANES = 50000 // _U_WORKERS


def _tc_binary_copy(b_hbm, ob_hbm, bv, sin, sout):
    for i in range(_NCHUNK):
        pltpu.make_async_copy(
            b_hbm.at[:, pl.ds(i * _CHUNK, _CHUNK)], bv.at[i], sin.at[i]
        ).start()
    outs = []
    for i in range(_NCHUNK):
        pltpu.make_async_copy(
            b_hbm.at[:, pl.ds(i * _CHUNK, _CHUNK)], bv.at[i], sin.at[i]
        ).wait()
        c = pltpu.make_async_copy(
            bv.at[i], ob_hbm.at[:, pl.ds(i * _CHUNK, _CHUNK)], sout.at[i]
        )
        c.start()
        outs.append(c)
    for c in outs:
        c.wait()


def _sc_unary_copy(uT):
    mesh = plsc.VectorSubcoreMesh(core_axis_name="c", subcore_axis_name="s")

    @functools.partial(
        pl.kernel,
        mesh=mesh,
        out_type=jax.ShapeDtypeStruct((8, 50000), jnp.float32),
    )
    def k(u_hbm, ou_hbm):
        wid = lax.axis_index("s") * 2 + lax.axis_index("c")

        @pl.when(wid == 0)
        def _():
            pltpu.sync_copy(u_hbm, ou_hbm)

    return k(uT)


def kernel(unary, binary, index1, index2):
    uT = unary.T
    bT = binary.T
    ouT = _sc_unary_copy(uT)
    obT = pl.pallas_call(
        _tc_binary_copy,
        in_specs=[pl.BlockSpec(memory_space=pl.ANY)],
        out_specs=pl.BlockSpec(memory_space=pl.ANY),
        out_shape=jax.ShapeDtypeStruct(bT.shape, bT.dtype),
        scratch_shapes=[
            pltpu.VMEM((_NCHUNK, 2, _CHUNK), jnp.float32),
            pltpu.SemaphoreType.DMA((_NCHUNK,)),
            pltpu.SemaphoreType.DMA((_NCHUNK,)),
        ],
    )(bT)
    return ouT.T, obT.T


# speedup vs baseline: 7.2455x; 7.2455x over previous
import jax
import jax.numpy as jnp
from jax.experimental import pallas as pl
from jax.experimental.pallas import tpu as pltpu

_NCHUNK = 2
_CHUNK = 1600000 // _NCHUNK


def _dma_kernel(u_hbm, b_hbm, ou_hbm, ob_hbm, uv, bv, su, so_u, sin, sout):
    pltpu.make_async_copy(
        b_hbm.at[:, pl.ds(0, _CHUNK)], bv.at[0], sin.at[0]
    ).start()
    cu_in = pltpu.make_async_copy(u_hbm, uv, su)
    cu_in.start()
    for i in range(1, _NCHUNK):
        pltpu.make_async_copy(
            b_hbm.at[:, pl.ds(i * _CHUNK, _CHUNK)], bv.at[i], sin.at[i]
        ).start()
    cu_in.wait()
    cu_out = pltpu.make_async_copy(uv, ou_hbm, so_u)
    cu_out.start()
    outs = []
    for i in range(_NCHUNK):
        pltpu.make_async_copy(
            b_hbm.at[:, pl.ds(i * _CHUNK, _CHUNK)], bv.at[i], sin.at[i]
        ).wait()
        c = pltpu.make_async_copy(
            bv.at[i], ob_hbm.at[:, pl.ds(i * _CHUNK, _CHUNK)], sout.at[i]
        )
        c.start()
        outs.append(c)
    cu_out.wait()
    for c in outs:
        c.wait()


def kernel(unary, binary, index1, index2):
    uT = unary.T
    bT = binary.T
    ouT, obT = pl.pallas_call(
        _dma_kernel,
        in_specs=[
            pl.BlockSpec(memory_space=pl.ANY),
            pl.BlockSpec(memory_space=pl.ANY),
        ],
        out_specs=[
            pl.BlockSpec(memory_space=pl.ANY),
            pl.BlockSpec(memory_space=pl.ANY),
        ],
        out_shape=[
            jax.ShapeDtypeStruct(uT.shape, uT.dtype),
            jax.ShapeDtypeStruct(bT.shape, bT.dtype),
        ],
        scratch_shapes=[
            pltpu.VMEM((8, 50000), jnp.float32),
            pltpu.VMEM((_NCHUNK, 2, _CHUNK), jnp.float32),
            pltpu.SemaphoreType.DMA,
            pltpu.SemaphoreType.DMA,
            pltpu.SemaphoreType.DMA((_NCHUNK,)),
            pltpu.SemaphoreType.DMA((_NCHUNK,)),
        ],
    )(uT, bT)
    return ouT.T, obT.T
